# Initial kernel scaffold; baseline (speedup 1.0000x reference)
import jax
import jax.numpy as jnp
from jax.experimental import pallas as pl


def kernel(rpn_scores, rpn_bbox_delta, anchors):
    raise NotImplementedError("write your pallas kernel here")


# speedup vs baseline: 50.6855x; 50.6855x over previous
import functools

import jax
import jax.numpy as jnp
from jax.experimental import pallas as pl
from jax.experimental.pallas import tpu as pltpu

_B = 2
_N = 20000
_R = 160
_C = 128
_K = 6000
_MAX_OUT = 1000
_TH = 0.7
_STD = (0.1, 0.1, 0.2, 0.2)
_ONE_BITS = 0x3F800000


def _nms_body(s_ref, d0_ref, d1_ref, d2_ref, d3_ref,
              a0_ref, a1_ref, a2_ref, a3_ref, out_ref,
              sc_ref, y1_ref, x1_ref, y2_ref, x2_ref):
    s = s_ref[0]

    ay1, ax1, ay2, ax2 = a0_ref[...], a1_ref[...], a2_ref[...], a3_ref[...]
    h = ay2 - ay1
    w = ax2 - ax1
    cy = ay1 + 0.5 * h
    cx = ax1 + 0.5 * w
    cy = cy + (d0_ref[0] * _STD[0]) * h
    cx = cx + (d1_ref[0] * _STD[1]) * w
    h = h * jnp.exp(d2_ref[0] * _STD[2])
    w = w * jnp.exp(d3_ref[0] * _STD[3])
    y1 = cy - 0.5 * h
    x1 = cx - 0.5 * w
    y2 = y1 + h
    x2 = x1 + w
    y1 = jnp.clip(y1, 0.0, 1.0)
    x1 = jnp.clip(x1, 0.0, 1.0)
    y2 = jnp.clip(y2, 0.0, 1.0)
    x2 = jnp.clip(x2, 0.0, 1.0)
    area = (y2 - y1) * (x2 - x1)
    y1_ref[...] = y1
    x1_ref[...] = x1
    y2_ref[...] = y2
    x2_ref[...] = x2

    bits = jax.lax.bitcast_convert_type(s, jnp.int32)

    def _bis(_, lh):
        lo, hi = lh
        mid = (lo + hi) // 2
        cge = jnp.sum((bits >= mid).astype(jnp.float32))
        take = cge >= float(_K)
        return (jnp.where(take, mid, lo), jnp.where(take, hi, mid))

    lo, _ = jax.lax.fori_loop(0, 31, _bis, (jnp.int32(0), jnp.int32(_ONE_BITS)))

    gt = bits > lo
    eq = bits == lo
    quota = float(_K) - jnp.sum(gt.astype(jnp.float32))
    li = jax.lax.broadcasted_iota(jnp.int32, (_C, _C), 0)
    lj = jax.lax.broadcasted_iota(jnp.int32, (_C, _C), 1)
    tri_incl = (li <= lj).astype(jnp.float32)
    ri = jax.lax.broadcasted_iota(jnp.int32, (_R, _R), 0)
    rj = jax.lax.broadcasted_iota(jnp.int32, (_R, _R), 1)
    tri_strict = (rj < ri).astype(jnp.float32)
    eqf = eq.astype(jnp.float32)
    within = jnp.dot(eqf, tri_incl, preferred_element_type=jnp.float32)
    rowtot = within[:, _C - 1:_C]
    rowpref = jnp.dot(tri_strict, rowtot,
                      preferred_element_type=jnp.float32)
    rank_incl = within + rowpref
    cand = jnp.logical_or(gt, jnp.logical_and(eq, rank_incl <= quota))
    sc_ref[...] = jnp.where(cand, s, -1.0)

    out_ref[0, :, :] = jnp.zeros((_MAX_OUT, _C), jnp.float32)

    fi = (jax.lax.broadcasted_iota(jnp.int32, (_R, _C), 0) * _C
          + jax.lax.broadcasted_iota(jnp.int32, (_R, _C), 1))
    fif = fi.astype(jnp.float32)
    lane = jax.lax.broadcasted_iota(jnp.int32, (1, _C), 1)

    def _round(_, cnt):
        scur = sc_ref[...]
        m = jnp.max(scur)
        sel_f = jnp.min(jnp.where(scur == m, fif, 3.0e7))
        imin = sel_f.astype(jnp.int32)
        r = imin // _C
        c = imin % _C
        onehot = (lane == c).astype(jnp.float32)
        by1 = jnp.sum(y1_ref[pl.ds(r, 1), :] * onehot)
        bx1 = jnp.sum(x1_ref[pl.ds(r, 1), :] * onehot)
        by2 = jnp.sum(y2_ref[pl.ds(r, 1), :] * onehot)
        bx2 = jnp.sum(x2_ref[pl.ds(r, 1), :] * onehot)
        p = m > -0.5
        pf = p.astype(jnp.float32)
        yy1 = jnp.maximum(y1_ref[...], by1)
        xx1 = jnp.maximum(x1_ref[...], bx1)
        yy2 = jnp.minimum(y2_ref[...], by2)
        xx2 = jnp.minimum(x2_ref[...], bx2)
        inter = jnp.maximum(yy2 - yy1, 0.0) * jnp.maximum(xx2 - xx1, 0.0)
        union = area + (by2 - by1) * (bx2 - bx1) - inter
        iou = jnp.where(union > 0.0, inter / union, 0.0)
        kill = jnp.logical_or(iou > _TH, fi == imin)
        sc_ref[...] = jnp.where(kill, -1.0, scur)
        val = jnp.where(lane == 0, by1,
              jnp.where(lane == 1, bx1,
              jnp.where(lane == 2, by2,
              jnp.where(lane == 3, bx2, 0.0)))) * pf
        out_ref[0, pl.ds(cnt, 1), :] = val
        return cnt + p.astype(jnp.int32)

    jax.lax.fori_loop(0, _MAX_OUT, _round, jnp.int32(0))


@jax.jit
def kernel(rpn_scores, rpn_bbox_delta, anchors):
    pad = _R * _C - _N

    def _planes(x3, pad_val):
        xp = jnp.pad(x3, ((0, 0), (0, pad), (0, 0)), constant_values=pad_val)
        return [xp[:, :, k].reshape(_B, _R, _C) for k in range(4)]

    scores = jnp.pad(rpn_scores[:, :, 1], ((0, 0), (0, pad)),
                     constant_values=-1.0).reshape(_B, _R, _C)
    d0, d1, d2, d3 = _planes(rpn_bbox_delta, 0.0)
    ap = jnp.pad(anchors, ((0, pad), (0, 0)))
    a0, a1, a2, a3 = [ap[:, k].reshape(_R, _C) for k in range(4)]

    bspec = pl.BlockSpec((1, _R, _C), lambda b: (b, 0, 0))
    aspec = pl.BlockSpec((_R, _C), lambda b: (0, 0))
    out = pl.pallas_call(
        _nms_body,
        grid=(_B,),
        in_specs=[bspec, bspec, bspec, bspec, bspec, aspec, aspec, aspec, aspec],
        out_specs=pl.BlockSpec((1, _MAX_OUT, _C), lambda b: (b, 0, 0)),
        out_shape=jax.ShapeDtypeStruct((_B, _MAX_OUT, _C), jnp.float32),
        scratch_shapes=[pltpu.VMEM((_R, _C), jnp.float32)] * 5,
    )(scores, d0, d1, d2, d3, a0, a1, a2, a3)
    return out[:, :, :4]


# speedup vs baseline: 58.7507x; 1.1591x over previous
import jax
import jax.numpy as jnp
from jax.experimental import pallas as pl
from jax.experimental.pallas import tpu as pltpu

_B = 2
_N = 20000
_R = 160
_C = 128
_K = 6000
_MAX_OUT = 1000
_TH = 0.7
_STD = (0.1, 0.1, 0.2, 0.2)
_ONE_BITS = 0x3F800000


def _nms_body(s_ref, d0_ref, d1_ref, d2_ref, d3_ref,
              a0_ref, a1_ref, a2_ref, a3_ref, out_ref,
              sc_ref, y1_ref, x1_ref, y2_ref, x2_ref):
    ay1, ax1, ay2, ax2 = a0_ref[...], a1_ref[...], a2_ref[...], a3_ref[...]
    ah = ay2 - ay1
    aw = ax2 - ax1
    acy = ay1 + 0.5 * ah
    acx = ax1 + 0.5 * aw

    li = jax.lax.broadcasted_iota(jnp.int32, (_C, _C), 0)
    lj = jax.lax.broadcasted_iota(jnp.int32, (_C, _C), 1)
    tri_incl = (li <= lj).astype(jnp.float32)
    ri = jax.lax.broadcasted_iota(jnp.int32, (_R, _R), 0)
    rj = jax.lax.broadcasted_iota(jnp.int32, (_R, _R), 1)
    tri_strict = (rj < ri).astype(jnp.float32)

    areas = []
    for b in range(_B):
        s = s_ref[b]
        cy = acy + (d0_ref[b] * _STD[0]) * ah
        cx = acx + (d1_ref[b] * _STD[1]) * aw
        h = ah * jnp.exp(d2_ref[b] * _STD[2])
        w = aw * jnp.exp(d3_ref[b] * _STD[3])
        y1 = jnp.clip(cy - 0.5 * h, 0.0, 1.0)
        x1 = jnp.clip(cx - 0.5 * w, 0.0, 1.0)
        y2 = jnp.clip((cy - 0.5 * h) + h, 0.0, 1.0)
        x2 = jnp.clip((cx - 0.5 * w) + w, 0.0, 1.0)
        areas.append((y2 - y1) * (x2 - x1))
        y1_ref[b] = y1
        x1_ref[b] = x1
        y2_ref[b] = y2
        x2_ref[b] = x2

        bits = jax.lax.bitcast_convert_type(s, jnp.int32)

        def _bis(_, lh):
            lo, hi = lh
            mid = (lo + hi) // 2
            cge = jnp.sum((bits >= mid).astype(jnp.float32))
            take = cge >= float(_K)
            return (jnp.where(take, mid, lo), jnp.where(take, hi, mid))

        lo, _ = jax.lax.fori_loop(0, 31, _bis,
                                  (jnp.int32(0), jnp.int32(_ONE_BITS)))
        gt = bits > lo
        eq = bits == lo
        quota = float(_K) - jnp.sum(gt.astype(jnp.float32))
        eqf = eq.astype(jnp.float32)
        within = jnp.dot(eqf, tri_incl, preferred_element_type=jnp.float32)
        rowtot = within[:, _C - 1:_C]
        rowpref = jnp.dot(tri_strict, rowtot,
                          preferred_element_type=jnp.float32)
        rank_incl = within + rowpref
        cand = jnp.logical_or(gt, jnp.logical_and(eq, rank_incl <= quota))
        sc_ref[b] = jnp.where(cand, s, -1.0)

    out_ref[...] = jnp.zeros((_B, _MAX_OUT, _C), jnp.float32)

    fi = (jax.lax.broadcasted_iota(jnp.int32, (_R, _C), 0) * _C
          + jax.lax.broadcasted_iota(jnp.int32, (_R, _C), 1))
    fif = fi.astype(jnp.float32)
    lane = jax.lax.broadcasted_iota(jnp.int32, (1, _C), 1)

    def _round_one(b, cnt):
        scur = sc_ref[b]
        m = jnp.max(scur)
        sel_f = jnp.min(jnp.where(scur == m, fif, 3.0e7))
        imin = sel_f.astype(jnp.int32)
        r = imin // _C
        c = imin % _C
        onehot = (lane == c).astype(jnp.float32)
        by1 = jnp.sum(y1_ref[b, pl.ds(r, 1), :] * onehot)
        bx1 = jnp.sum(x1_ref[b, pl.ds(r, 1), :] * onehot)
        by2 = jnp.sum(y2_ref[b, pl.ds(r, 1), :] * onehot)
        bx2 = jnp.sum(x2_ref[b, pl.ds(r, 1), :] * onehot)
        p = m > -0.5
        pf = p.astype(jnp.float32)
        yy1 = jnp.maximum(y1_ref[b], by1)
        xx1 = jnp.maximum(x1_ref[b], bx1)
        yy2 = jnp.minimum(y2_ref[b], by2)
        xx2 = jnp.minimum(x2_ref[b], bx2)
        inter = jnp.maximum(yy2 - yy1, 0.0) * jnp.maximum(xx2 - xx1, 0.0)
        union = areas[b] + (by2 - by1) * (bx2 - bx1) - inter
        iou = jnp.where(union > 0.0, inter / union, 0.0)
        kill = jnp.logical_or(iou > _TH, fi == imin)
        sc_ref[b] = jnp.where(kill, -1.0, scur)
        val = jnp.where(lane == 0, by1,
              jnp.where(lane == 1, bx1,
              jnp.where(lane == 2, by2,
              jnp.where(lane == 3, bx2, 0.0)))) * pf
        out_ref[b, pl.ds(cnt, 1), :] = val
        return cnt + p.astype(jnp.int32)

    def _round(_, cnts):
        return tuple(_round_one(b, cnts[b]) for b in range(_B))

    jax.lax.fori_loop(0, _MAX_OUT, _round, (jnp.int32(0),) * _B)


@jax.jit
def kernel(rpn_scores, rpn_bbox_delta, anchors):
    pad = _R * _C - _N

    def _planes(x3, pad_val):
        xp = jnp.pad(x3, ((0, 0), (0, pad), (0, 0)), constant_values=pad_val)
        return [xp[:, :, k].reshape(_B, _R, _C) for k in range(4)]

    scores = jnp.pad(rpn_scores[:, :, 1], ((0, 0), (0, pad)),
                     constant_values=-1.0).reshape(_B, _R, _C)
    d0, d1, d2, d3 = _planes(rpn_bbox_delta, 0.0)
    ap = jnp.pad(anchors, ((0, pad), (0, 0)))
    a0, a1, a2, a3 = [ap[:, k].reshape(_R, _C) for k in range(4)]

    out = pl.pallas_call(
        _nms_body,
        out_shape=jax.ShapeDtypeStruct((_B, _MAX_OUT, _C), jnp.float32),
        scratch_shapes=[pltpu.VMEM((_B, _R, _C), jnp.float32)] * 5,
    )(scores, d0, d1, d2, d3, a0, a1, a2, a3)
    return out[:, :, :4]


# speedup vs baseline: 59.0241x; 1.0047x over previous
import jax
import jax.numpy as jnp
from jax.experimental import pallas as pl
from jax.experimental.pallas import tpu as pltpu

_B = 2
_N = 20000
_R = 160
_C = 128
_K = 6000
_MAX_OUT = 1000
_TH = 0.7
_STD = (0.1, 0.1, 0.2, 0.2)
_ONE_BITS = 0x3F800000


def _nms_body(s_ref, d0_ref, d1_ref, d2_ref, d3_ref,
              a0_ref, a1_ref, a2_ref, a3_ref, out_ref,
              sc_ref, y1_ref, x1_ref, y2_ref, x2_ref):
    ay1, ax1, ay2, ax2 = a0_ref[...], a1_ref[...], a2_ref[...], a3_ref[...]
    ah = ay2 - ay1
    aw = ax2 - ax1
    acy = ay1 + 0.5 * ah
    acx = ax1 + 0.5 * aw

    li = jax.lax.broadcasted_iota(jnp.int32, (_C, _C), 0)
    lj = jax.lax.broadcasted_iota(jnp.int32, (_C, _C), 1)
    tri_incl = (li <= lj).astype(jnp.float32)
    ri = jax.lax.broadcasted_iota(jnp.int32, (_R, _R), 0)
    rj = jax.lax.broadcasted_iota(jnp.int32, (_R, _R), 1)
    tri_strict = (rj < ri).astype(jnp.float32)

    areas = []
    for b in range(_B):
        s = s_ref[b]
        cy = acy + (d0_ref[b] * _STD[0]) * ah
        cx = acx + (d1_ref[b] * _STD[1]) * aw
        h = ah * jnp.exp(d2_ref[b] * _STD[2])
        w = aw * jnp.exp(d3_ref[b] * _STD[3])
        y1 = jnp.clip(cy - 0.5 * h, 0.0, 1.0)
        x1 = jnp.clip(cx - 0.5 * w, 0.0, 1.0)
        y2 = jnp.clip((cy - 0.5 * h) + h, 0.0, 1.0)
        x2 = jnp.clip((cx - 0.5 * w) + w, 0.0, 1.0)
        areas.append((y2 - y1) * (x2 - x1))
        y1_ref[b] = y1
        x1_ref[b] = x1
        y2_ref[b] = y2
        x2_ref[b] = x2

        bits = jax.lax.bitcast_convert_type(s, jnp.int32)

        def _bis(_, lh):
            lo, hi = lh
            mid = (lo + hi) // 2
            cge = jnp.sum((bits >= mid).astype(jnp.float32))
            take = cge >= float(_K)
            return (jnp.where(take, mid, lo), jnp.where(take, hi, mid))

        lo, _ = jax.lax.fori_loop(0, 31, _bis,
                                  (jnp.int32(0), jnp.int32(_ONE_BITS)))
        gt = bits > lo
        eq = bits == lo
        quota = float(_K) - jnp.sum(gt.astype(jnp.float32))
        eqf = eq.astype(jnp.float32)
        within = jnp.dot(eqf, tri_incl, preferred_element_type=jnp.float32)
        rowtot = within[:, _C - 1:_C]
        rowpref = jnp.dot(tri_strict, rowtot,
                          preferred_element_type=jnp.float32)
        rank_incl = within + rowpref
        cand = jnp.logical_or(gt, jnp.logical_and(eq, rank_incl <= quota))
        sc_ref[b] = jnp.where(cand, s, -1.0)

    out_ref[...] = jnp.zeros((_B, _MAX_OUT, _C), jnp.float32)

    fi = (jax.lax.broadcasted_iota(jnp.int32, (_R, _C), 0) * _C
          + jax.lax.broadcasted_iota(jnp.int32, (_R, _C), 1))
    fif = fi.astype(jnp.float32)
    lane = jax.lax.broadcasted_iota(jnp.int32, (1, _C), 1)

    def _extract(b, sel_f):
        imin = sel_f.astype(jnp.int32)
        r = imin // _C
        c = imin % _C
        onehot = (lane == c).astype(jnp.float32)
        by1 = jnp.sum(y1_ref[b, pl.ds(r, 1), :] * onehot)
        bx1 = jnp.sum(x1_ref[b, pl.ds(r, 1), :] * onehot)
        by2 = jnp.sum(y2_ref[b, pl.ds(r, 1), :] * onehot)
        bx2 = jnp.sum(x2_ref[b, pl.ds(r, 1), :] * onehot)
        return by1, bx1, by2, bx2

    def _iou_vs_all(b, box):
        by1, bx1, by2, bx2 = box
        yy1 = jnp.maximum(y1_ref[b], by1)
        xx1 = jnp.maximum(x1_ref[b], bx1)
        yy2 = jnp.minimum(y2_ref[b], by2)
        xx2 = jnp.minimum(x2_ref[b], bx2)
        inter = jnp.maximum(yy2 - yy1, 0.0) * jnp.maximum(xx2 - xx1, 0.0)
        union = areas[b] + (by2 - by1) * (bx2 - bx1) - inter
        return jnp.where(union > 0.0, inter / union, 0.0)

    def _iou_scalar(box_a, box_b):
        ay1_, ax1_, ay2_, ax2_ = box_a
        by1_, bx1_, by2_, bx2_ = box_b
        yy1 = jnp.maximum(ay1_, by1_)
        xx1 = jnp.maximum(ax1_, bx1_)
        yy2 = jnp.minimum(ay2_, by2_)
        xx2 = jnp.minimum(ax2_, bx2_)
        inter = jnp.maximum(yy2 - yy1, 0.0) * jnp.maximum(xx2 - xx1, 0.0)
        area_a = (ay2_ - ay1_) * (ax2_ - ax1_)
        area_b = (by2_ - by1_) * (bx2_ - bx1_)
        union = area_a + area_b - inter
        return jnp.where(union > 0.0, inter / union, 0.0)

    def _store(b, cnt, box):
        by1, bx1, by2, bx2 = box
        val = jnp.where(lane == 0, by1,
              jnp.where(lane == 1, bx1,
              jnp.where(lane == 2, by2,
              jnp.where(lane == 3, bx2, 0.0))))
        out_ref[b, pl.ds(cnt, 1), :] = val

    def _round_one(b, cnt):
        scur = sc_ref[b]
        m1 = jnp.max(scur)
        f1 = jnp.min(jnp.where(scur == m1, fif, 3.0e7))
        sel1 = fif == f1
        scur1 = jnp.where(sel1, -1.0, scur)
        m2 = jnp.max(scur1)
        f2 = jnp.min(jnp.where(scur1 == m2, fif, 3.0e7))
        sel2 = fif == f2
        box1 = _extract(b, f1)
        box2 = _extract(b, f2)
        take1 = jnp.logical_and(m1 > -0.5, cnt < _MAX_OUT)
        t1i = take1.astype(jnp.int32)
        take2 = jnp.logical_and(
            jnp.logical_and(m2 > -0.5, _iou_scalar(box1, box2) <= _TH),
            cnt + t1i < _MAX_OUT)
        t2i = take2.astype(jnp.int32)
        iou1 = _iou_vs_all(b, box1)
        iou2 = _iou_vs_all(b, box2)
        kill = jnp.logical_or(
            iou1 > _TH,
            jnp.logical_and(take2, jnp.logical_or(iou2 > _TH, sel2)))
        sc_ref[b] = jnp.where(kill, -1.0, scur1)

        @pl.when(take1)
        def _():
            _store(b, cnt, box1)

        @pl.when(take2)
        def _():
            _store(b, cnt + t1i, box2)

        return cnt + t1i + t2i, m1 > -0.5

    def _cond(state):
        cnt0, cnt1, al0, al1 = state
        return jnp.logical_or(jnp.logical_and(al0, cnt0 < _MAX_OUT),
                              jnp.logical_and(al1, cnt1 < _MAX_OUT))

    def _round(state):
        cnt0, cnt1, _, _ = state
        cnt0, al0 = _round_one(0, cnt0)
        cnt1, al1 = _round_one(1, cnt1)
        return (cnt0, cnt1, al0, al1)

    jax.lax.while_loop(_cond, _round,
                       (jnp.int32(0), jnp.int32(0), True, True))


@jax.jit
def kernel(rpn_scores, rpn_bbox_delta, anchors):
    pad = _R * _C - _N

    def _planes(x3, pad_val):
        xp = jnp.pad(x3, ((0, 0), (0, pad), (0, 0)), constant_values=pad_val)
        return [xp[:, :, k].reshape(_B, _R, _C) for k in range(4)]

    scores = jnp.pad(rpn_scores[:, :, 1], ((0, 0), (0, pad)),
                     constant_values=-1.0).reshape(_B, _R, _C)
    d0, d1, d2, d3 = _planes(rpn_bbox_delta, 0.0)
    ap = jnp.pad(anchors, ((0, pad), (0, 0)))
    a0, a1, a2, a3 = [ap[:, k].reshape(_R, _C) for k in range(4)]

    out = pl.pallas_call(
        _nms_body,
        out_shape=jax.ShapeDtypeStruct((_B, _MAX_OUT, _C), jnp.float32),
        scratch_shapes=[pltpu.VMEM((_B, _R, _C), jnp.float32)] * 5,
    )(scores, d0, d1, d2, d3, a0, a1, a2, a3)
    return out[:, :, :4]


# speedup vs baseline: 59.5320x; 1.0086x over previous
import jax
import jax.numpy as jnp
from jax.experimental import pallas as pl
from jax.experimental.pallas import tpu as pltpu

_B = 2
_N = 20000
_R = 160
_C = 128
_K = 6000
_MAX_OUT = 1000
_TH = 0.7
_STD = (0.1, 0.1, 0.2, 0.2)
_ONE_BITS = 0x3F800000


def _nms_body(s_ref, d0_ref, d1_ref, d2_ref, d3_ref,
              a0_ref, a1_ref, a2_ref, a3_ref, out_ref,
              sc_ref, y1_ref, x1_ref, y2_ref, x2_ref):
    ay1, ax1, ay2, ax2 = a0_ref[...], a1_ref[...], a2_ref[...], a3_ref[...]
    ah = ay2 - ay1
    aw = ax2 - ax1
    acy = ay1 + 0.5 * ah
    acx = ax1 + 0.5 * aw

    li = jax.lax.broadcasted_iota(jnp.int32, (_C, _C), 0)
    lj = jax.lax.broadcasted_iota(jnp.int32, (_C, _C), 1)
    tri_incl = (li <= lj).astype(jnp.float32)
    ri = jax.lax.broadcasted_iota(jnp.int32, (_R, _R), 0)
    rj = jax.lax.broadcasted_iota(jnp.int32, (_R, _R), 1)
    tri_strict = (rj < ri).astype(jnp.float32)

    areas = []
    for b in range(_B):
        s = s_ref[b]
        cy = acy + (d0_ref[b] * _STD[0]) * ah
        cx = acx + (d1_ref[b] * _STD[1]) * aw
        h = ah * jnp.exp(d2_ref[b] * _STD[2])
        w = aw * jnp.exp(d3_ref[b] * _STD[3])
        y1 = jnp.clip(cy - 0.5 * h, 0.0, 1.0)
        x1 = jnp.clip(cx - 0.5 * w, 0.0, 1.0)
        y2 = jnp.clip((cy - 0.5 * h) + h, 0.0, 1.0)
        x2 = jnp.clip((cx - 0.5 * w) + w, 0.0, 1.0)
        areas.append((y2 - y1) * (x2 - x1))
        y1_ref[b] = y1
        x1_ref[b] = x1
        y2_ref[b] = y2
        x2_ref[b] = x2

        bits = jax.lax.bitcast_convert_type(s, jnp.int32)

        def _bis(_, lh):
            lo, hi = lh
            mid = (lo + hi) // 2
            cge = jnp.sum((bits >= mid).astype(jnp.float32))
            take = cge >= float(_K)
            return (jnp.where(take, mid, lo), jnp.where(take, hi, mid))

        lo, _ = jax.lax.fori_loop(0, 31, _bis,
                                  (jnp.int32(0), jnp.int32(_ONE_BITS)))
        gt = bits > lo
        eq = bits == lo
        quota = float(_K) - jnp.sum(gt.astype(jnp.float32))
        eqf = eq.astype(jnp.float32)
        within = jnp.dot(eqf, tri_incl, preferred_element_type=jnp.float32)
        rowtot = within[:, _C - 1:_C]
        rowpref = jnp.dot(tri_strict, rowtot,
                          preferred_element_type=jnp.float32)
        rank_incl = within + rowpref
        cand = jnp.logical_or(gt, jnp.logical_and(eq, rank_incl <= quota))
        sc_ref[b] = jnp.where(cand, s, -1.0)

    out_ref[...] = jnp.zeros((_B, _MAX_OUT, _C), jnp.float32)

    fif = (jax.lax.broadcasted_iota(jnp.int32, (_R, _C), 0) * _C
           + jax.lax.broadcasted_iota(jnp.int32, (_R, _C), 1)
           ).astype(jnp.float32)
    lane_i = jax.lax.broadcasted_iota(jnp.int32, (1, _C), 1)
    lane = lane_i.astype(jnp.float32)

    def _round_one(b, cnt):
        scur = sc_ref[b]
        m = jnp.max(scur, keepdims=True)
        f1 = jnp.min(jnp.where(scur == m, fif, 3.0e7), keepdims=True)
        r = f1[0, 0].astype(jnp.int32) // _C
        cv = f1 - jnp.floor(f1 * (1.0 / _C)) * float(_C)
        onehot = (lane == cv).astype(jnp.float32)
        by1 = jnp.sum(y1_ref[b, pl.ds(r, 1), :] * onehot, keepdims=True)
        bx1 = jnp.sum(x1_ref[b, pl.ds(r, 1), :] * onehot, keepdims=True)
        by2 = jnp.sum(y2_ref[b, pl.ds(r, 1), :] * onehot, keepdims=True)
        bx2 = jnp.sum(x2_ref[b, pl.ds(r, 1), :] * onehot, keepdims=True)
        yy1 = jnp.maximum(y1_ref[b], by1)
        xx1 = jnp.maximum(x1_ref[b], bx1)
        yy2 = jnp.minimum(y2_ref[b], by2)
        xx2 = jnp.minimum(x2_ref[b], bx2)
        inter = jnp.maximum(yy2 - yy1, 0.0) * jnp.maximum(xx2 - xx1, 0.0)
        union = areas[b] + (by2 - by1) * (bx2 - bx1) - inter
        iou = jnp.where(union > 0.0, inter / union, 0.0)
        kill = jnp.logical_or(iou > _TH, fif == f1)
        sc_ref[b] = jnp.where(kill, -1.0, scur)
        pv = (m > -0.5).astype(jnp.float32)
        val = jnp.where(lane_i == 0, by1,
              jnp.where(lane_i == 1, bx1,
              jnp.where(lane_i == 2, by2,
              jnp.where(lane_i == 3, bx2, 0.0)))) * pv
        out_ref[b, pl.ds(cnt, 1), :] = val
        return cnt + (m[0, 0] > -0.5).astype(jnp.int32)

    def _round(_, cnts):
        return tuple(_round_one(b, cnts[b]) for b in range(_B))

    jax.lax.fori_loop(0, _MAX_OUT, _round, (jnp.int32(0),) * _B)


@jax.jit
def kernel(rpn_scores, rpn_bbox_delta, anchors):
    pad = _R * _C - _N

    def _planes(x3, pad_val):
        xp = jnp.pad(x3, ((0, 0), (0, pad), (0, 0)), constant_values=pad_val)
        return [xp[:, :, k].reshape(_B, _R, _C) for k in range(4)]

    scores = jnp.pad(rpn_scores[:, :, 1], ((0, 0), (0, pad)),
                     constant_values=-1.0).reshape(_B, _R, _C)
    d0, d1, d2, d3 = _planes(rpn_bbox_delta, 0.0)
    ap = jnp.pad(anchors, ((0, pad), (0, 0)))
    a0, a1, a2, a3 = [ap[:, k].reshape(_R, _C) for k in range(4)]

    out = pl.pallas_call(
        _nms_body,
        out_shape=jax.ShapeDtypeStruct((_B, _MAX_OUT, _C), jnp.float32),
        scratch_shapes=[pltpu.VMEM((_B, _R, _C), jnp.float32)] * 5,
    )(scores, d0, d1, d2, d3, a0, a1, a2, a3)
    return out[:, :, :4]
